# baseline (speedup 1.0000x reference)
import functools

import jax
import jax.numpy as jnp
from jax import lax
from jax.experimental import pallas as pl
from jax.experimental.pallas import tpu as pltpu
from jax.experimental.pallas import tpu_sc as plsc

N = 10000
E = 320000
F = 128
R = 8
NB = 4

NC = 2
NS = 16
NW = NC * NS

EPR = 2560
EP = EPR * F
RPW = EPR // NW
PNL = 16
NPAD = 10112
ROWS_PER_TILE = NPAD // NS

XB = 5000


def _transform_body(x_ref, basis_ref, wc_ref, out_ref):
    x = x_ref[...]
    for r in range(R):
        W = (wc_ref[r, 0, 0] * basis_ref[0] + wc_ref[r, 0, 1] * basis_ref[1]
             + wc_ref[r, 0, 2] * basis_ref[2] + wc_ref[r, 0, 3] * basis_ref[3])
        out_ref[r] = jnp.dot(x, W, preferred_element_type=jnp.float32)


def _all_transform(node_feats, basis, w_comp):
    nblk = N // XB
    out = pl.pallas_call(
        _transform_body,
        grid=(nblk,),
        in_specs=[
            pl.BlockSpec((XB, F), lambda j: (j, 0)),
            pl.BlockSpec((NB, F, F), lambda j: (0, 0, 0)),
            pl.BlockSpec((R, 1, NB), lambda j: (0, 0, 0)),
        ],
        out_specs=pl.BlockSpec((R, XB, F), lambda j: (0, j, 0)),
        out_shape=jax.ShapeDtypeStruct((R, N, F), jnp.float32),
        compiler_params=pltpu.CompilerParams(
            dimension_semantics=("arbitrary",)),
    )(node_feats, basis, w_comp.reshape(R, 1, NB))
    return out.reshape(R * N, F)


def _keys_body(src_ref, et_ref, out_ref):
    out_ref[...] = et_ref[...] * N + src_ref[...]


def _edge_keys(src_p, et_p):
    return pl.pallas_call(
        _keys_body,
        out_shape=jax.ShapeDtypeStruct((EPR, F), jnp.int32),
    )(src_p, et_p)


def _sc_agg_body(table, keys_hbm, dst_hbm, out,
                 keys0, keys1, dst0, dst1, rows0, rows1, acc,
                 sem0, sem1, isk0, isk1, isd0, isd1):
    c = lax.axis_index("c")
    s = lax.axis_index("s")
    wid = c * NS + s
    base = wid * RPW
    kb, db = (keys0, keys1), (dst0, dst1)
    ksem, dsem = (isk0, isk1), (isd0, isd1)

    def load_panel(p, b):
        pltpu.async_copy(
            keys_hbm.at[pl.ds(base + p * PNL, PNL), :], kb[b], ksem[b])
        pltpu.async_copy(
            dst_hbm.at[pl.ds(base + p * PNL, PNL), :], db[b], dsem[b])

    def wait_panel(p, b):
        pltpu.make_async_copy(
            keys_hbm.at[pl.ds(base + p * PNL, PNL), :], kb[b], ksem[b]).wait()
        pltpu.make_async_copy(
            dst_hbm.at[pl.ds(base + p * PNL, PNL), :], db[b], dsem[b]).wait()

    def gather(kref, j, buf, sem):
        pltpu.async_copy(table.at[kref.at[j]], buf, sem)

    def gwait(kref, j, buf, sem):
        pltpu.make_async_copy(table.at[kref.at[j]], buf, sem).wait()

    load_panel(0, 0)
    load_panel(1, 1)

    def zero_body(i, carry):
        z = jnp.zeros((16,), jnp.float32)
        for k in range(8):
            rows0[i, pl.ds(k * 16, 16)] = z
        return carry

    lax.fori_loop(0, F, zero_body, 0, unroll=False)
    for k in range(ROWS_PER_TILE // F):
        pltpu.sync_copy(
            rows0, acc.at[pl.ds(s * ROWS_PER_TILE + k * F, F), :])
    rem = ROWS_PER_TILE % F
    if rem:
        pltpu.sync_copy(
            rows0.at[pl.ds(0, rem), :],
            acc.at[pl.ds(s * ROWS_PER_TILE + (ROWS_PER_TILE // F) * F, rem), :])

    wait_panel(0, 0)
    gather(kb[0], 0, rows0, sem0)
    plsc.subcore_barrier()

    NP = RPW // PNL
    for p in range(NP):
        K, D = kb[p % 2], db[p % 2]

        def body(j, carry, K=K, D=D):
            gather(K, 2 * j + 1, rows1, sem1)
            gwait(K, 2 * j, rows0, sem0)
            pltpu.sync_copy(rows0, acc.at[D.at[2 * j]], add=True)
            gather(K, 2 * j + 2, rows0, sem0)
            gwait(K, 2 * j + 1, rows1, sem1)
            pltpu.sync_copy(rows1, acc.at[D.at[2 * j + 1]], add=True)
            return carry

        lax.fori_loop(0, PNL // 2 - 1, body, 0, unroll=False)
        gather(K, PNL - 1, rows1, sem1)
        gwait(K, PNL - 2, rows0, sem0)
        pltpu.sync_copy(rows0, acc.at[D.at[PNL - 2]], add=True)
        if p + 1 < NP:
            wait_panel(p + 1, (p + 1) % 2)
            gather(kb[(p + 1) % 2], 0, rows0, sem0)
        gwait(K, PNL - 1, rows1, sem1)
        pltpu.sync_copy(rows1, acc.at[D.at[PNL - 1]], add=True)
        if p + 2 < NP:
            load_panel(p + 2, p % 2)

    plsc.subcore_barrier()
    pltpu.sync_copy(acc.at[pl.ds(s * ROWS_PER_TILE, ROWS_PER_TILE), :],
                    out.at[c, pl.ds(s * ROWS_PER_TILE, ROWS_PER_TILE), :])


@functools.cache
def _build_sc_agg():
    return pl.kernel(
        _sc_agg_body,
        out_type=jax.ShapeDtypeStruct((NC, NPAD, F), jnp.float32),
        mesh=plsc.VectorSubcoreMesh(core_axis_name="c", subcore_axis_name="s",
                                    num_cores=NC, num_subcores=NS),
        scratch_types=[
            pltpu.VMEM((PNL, F), jnp.int32),
            pltpu.VMEM((PNL, F), jnp.int32),
            pltpu.VMEM((PNL, F), jnp.int32),
            pltpu.VMEM((PNL, F), jnp.int32),
            pltpu.VMEM((F, F), jnp.float32),
            pltpu.VMEM((F, F), jnp.float32),
            pltpu.VMEM_SHARED((NPAD, F), jnp.float32),
            pltpu.SemaphoreType.DMA,
            pltpu.SemaphoreType.DMA,
            pltpu.SemaphoreType.DMA,
            pltpu.SemaphoreType.DMA,
            pltpu.SemaphoreType.DMA,
            pltpu.SemaphoreType.DMA,
        ],
    )


_SQRT1_2 = 0.7071067811865476


def _final_body(part_ref, x_ref, lw_ref, bias_ref, g_ref, b_ref, out_ref):
    x = x_ref[...]
    h = (part_ref[0] + part_ref[1]
         + jnp.dot(x, lw_ref[...], preferred_element_type=jnp.float32)
         + bias_ref[...] + x)
    g = 0.5 * h * (1.0 + lax.erf(h * _SQRT1_2))
    mean = jnp.mean(g, axis=-1, keepdims=True)
    cent = g - mean
    var = jnp.mean(cent * cent, axis=-1, keepdims=True)
    out_ref[...] = cent * lax.rsqrt(var + 1e-5) * g_ref[...] + b_ref[...]


def _finalize(partials, node_feats, loop_weight, bias, ln_gamma, ln_beta):
    nblk = N // XB
    return pl.pallas_call(
        _final_body,
        grid=(nblk,),
        in_specs=[
            pl.BlockSpec((NC, XB, F), lambda j: (0, j, 0)),
            pl.BlockSpec((XB, F), lambda j: (j, 0)),
            pl.BlockSpec((F, F), lambda j: (0, 0)),
            pl.BlockSpec((1, F), lambda j: (0, 0)),
            pl.BlockSpec((1, F), lambda j: (0, 0)),
            pl.BlockSpec((1, F), lambda j: (0, 0)),
        ],
        out_specs=pl.BlockSpec((XB, F), lambda j: (j, 0)),
        out_shape=jax.ShapeDtypeStruct((N, F), jnp.float32),
    )(partials, node_feats, loop_weight, bias.reshape(1, F),
      ln_gamma.reshape(1, F), ln_beta.reshape(1, F))


def kernel(node_feats, edge_index, edge_types, basis, w_comp, loop_weight,
           bias, ln_gamma, ln_beta):
    src = edge_index[0]
    dst = edge_index[1]
    pad = EP - E
    pad_iota = jnp.arange(pad, dtype=jnp.int32)
    src_p = jnp.concatenate([src, pad_iota % N]).reshape(EPR, F)
    et_p = jnp.concatenate([edge_types, jnp.zeros((pad,), jnp.int32)]
                           ).reshape(EPR, F)
    dst_p = jnp.concatenate([dst, N + pad_iota % (NPAD - N)]).reshape(EPR, F)

    all_t = _all_transform(node_feats, basis, w_comp)
    keys = _edge_keys(src_p, et_p)
    partials = _build_sc_agg()(all_t, keys, dst_p)
    return _finalize(partials, node_feats, loop_weight, bias, ln_gamma, ln_beta)


# speedup vs baseline: 1.0217x; 1.0217x over previous
import functools

import jax
import jax.numpy as jnp
from jax import lax
from jax.experimental import pallas as pl
from jax.experimental.pallas import tpu as pltpu
from jax.experimental.pallas import tpu_sc as plsc

N = 10000
E = 320000
F = 128
R = 8
NB = 4

NC = 2
NS = 16
NW = NC * NS

EPR = 2560
EP = EPR * F
RPW = EPR // NW
PNL = 16
NPAD = 10112
ROWS_PER_TILE = NPAD // NS

XB = 2000
KB = EPR // (N // XB)


def _transform_body(x_ref, basis_ref, wc_ref, src_ref, et_ref,
                    out_ref, keys_ref):
    x = x_ref[...]
    for r in range(R):
        W = (wc_ref[r, 0, 0] * basis_ref[0] + wc_ref[r, 0, 1] * basis_ref[1]
             + wc_ref[r, 0, 2] * basis_ref[2] + wc_ref[r, 0, 3] * basis_ref[3])
        out_ref[r] = jnp.dot(x, W, preferred_element_type=jnp.float32)
    keys_ref[...] = et_ref[...] * N + src_ref[...]


def _all_transform(node_feats, basis, w_comp, src_p, et_p):
    nblk = N // XB
    out, keys = pl.pallas_call(
        _transform_body,
        grid=(nblk,),
        in_specs=[
            pl.BlockSpec((XB, F), lambda j: (j, 0)),
            pl.BlockSpec((NB, F, F), lambda j: (0, 0, 0)),
            pl.BlockSpec((R, 1, NB), lambda j: (0, 0, 0)),
            pl.BlockSpec((KB, F), lambda j: (j, 0)),
            pl.BlockSpec((KB, F), lambda j: (j, 0)),
        ],
        out_specs=[
            pl.BlockSpec((R, XB, F), lambda j: (0, j, 0)),
            pl.BlockSpec((KB, F), lambda j: (j, 0)),
        ],
        out_shape=[
            jax.ShapeDtypeStruct((R, N, F), jnp.float32),
            jax.ShapeDtypeStruct((EPR, F), jnp.int32),
        ],
        compiler_params=pltpu.CompilerParams(
            dimension_semantics=("arbitrary",)),
    )(node_feats, basis, w_comp.reshape(R, 1, NB), src_p, et_p)
    return out.reshape(R * N, F), keys


def _sc_agg_body(table, keys_hbm, dst_hbm, out,
                 keys0, keys1, dst0, dst1, rows0, rows1, acc,
                 sem0, sem1, isk0, isk1, isd0, isd1):
    c = lax.axis_index("c")
    s = lax.axis_index("s")
    wid = c * NS + s
    base = wid * RPW
    kb, db = (keys0, keys1), (dst0, dst1)
    ksem, dsem = (isk0, isk1), (isd0, isd1)

    def load_panel(p, b):
        pltpu.async_copy(
            keys_hbm.at[pl.ds(base + p * PNL, PNL), :], kb[b], ksem[b])
        pltpu.async_copy(
            dst_hbm.at[pl.ds(base + p * PNL, PNL), :], db[b], dsem[b])

    def wait_panel(p, b):
        pltpu.make_async_copy(
            keys_hbm.at[pl.ds(base + p * PNL, PNL), :], kb[b], ksem[b]).wait()
        pltpu.make_async_copy(
            dst_hbm.at[pl.ds(base + p * PNL, PNL), :], db[b], dsem[b]).wait()

    def gather(kref, j, buf, sem):
        pltpu.async_copy(table.at[kref.at[j]], buf, sem)

    def gwait(kref, j, buf, sem):
        pltpu.make_async_copy(table.at[kref.at[j]], buf, sem).wait()

    load_panel(0, 0)
    load_panel(1, 1)

    def zero_body(i, carry):
        z = jnp.zeros((16,), jnp.float32)
        for k in range(8):
            rows0[i, pl.ds(k * 16, 16)] = z
        return carry

    lax.fori_loop(0, F, zero_body, 0, unroll=False)
    for k in range(ROWS_PER_TILE // F):
        pltpu.sync_copy(
            rows0, acc.at[pl.ds(s * ROWS_PER_TILE + k * F, F), :])
    rem = ROWS_PER_TILE % F
    if rem:
        pltpu.sync_copy(
            rows0.at[pl.ds(0, rem), :],
            acc.at[pl.ds(s * ROWS_PER_TILE + (ROWS_PER_TILE // F) * F, rem), :])

    wait_panel(0, 0)
    gather(kb[0], 0, rows0, sem0)
    plsc.subcore_barrier()

    NP = RPW // PNL
    for p in range(NP):
        K, D = kb[p % 2], db[p % 2]

        def body(j, carry, K=K, D=D):
            gather(K, 2 * j + 1, rows1, sem1)
            gwait(K, 2 * j, rows0, sem0)
            pltpu.sync_copy(rows0, acc.at[D.at[2 * j]], add=True)
            gather(K, 2 * j + 2, rows0, sem0)
            gwait(K, 2 * j + 1, rows1, sem1)
            pltpu.sync_copy(rows1, acc.at[D.at[2 * j + 1]], add=True)
            return carry

        lax.fori_loop(0, PNL // 2 - 1, body, 0, unroll=False)
        gather(K, PNL - 1, rows1, sem1)
        gwait(K, PNL - 2, rows0, sem0)
        pltpu.sync_copy(rows0, acc.at[D.at[PNL - 2]], add=True)
        if p + 1 < NP:
            wait_panel(p + 1, (p + 1) % 2)
            gather(kb[(p + 1) % 2], 0, rows0, sem0)
        gwait(K, PNL - 1, rows1, sem1)
        pltpu.sync_copy(rows1, acc.at[D.at[PNL - 1]], add=True)
        if p + 2 < NP:
            load_panel(p + 2, p % 2)

    plsc.subcore_barrier()
    pltpu.sync_copy(acc.at[pl.ds(s * ROWS_PER_TILE, ROWS_PER_TILE), :],
                    out.at[c, pl.ds(s * ROWS_PER_TILE, ROWS_PER_TILE), :])


@functools.cache
def _build_sc_agg():
    return pl.kernel(
        _sc_agg_body,
        out_type=jax.ShapeDtypeStruct((NC, NPAD, F), jnp.float32),
        mesh=plsc.VectorSubcoreMesh(core_axis_name="c", subcore_axis_name="s",
                                    num_cores=NC, num_subcores=NS),
        scratch_types=[
            pltpu.VMEM((PNL, F), jnp.int32),
            pltpu.VMEM((PNL, F), jnp.int32),
            pltpu.VMEM((PNL, F), jnp.int32),
            pltpu.VMEM((PNL, F), jnp.int32),
            pltpu.VMEM((F, F), jnp.float32),
            pltpu.VMEM((F, F), jnp.float32),
            pltpu.VMEM_SHARED((NPAD, F), jnp.float32),
            pltpu.SemaphoreType.DMA,
            pltpu.SemaphoreType.DMA,
            pltpu.SemaphoreType.DMA,
            pltpu.SemaphoreType.DMA,
            pltpu.SemaphoreType.DMA,
            pltpu.SemaphoreType.DMA,
        ],
    )


_SQRT1_2 = 0.7071067811865476


def _final_body(part_ref, x_ref, lw_ref, bias_ref, g_ref, b_ref, out_ref):
    x = x_ref[...]
    h = (part_ref[0] + part_ref[1]
         + jnp.dot(x, lw_ref[...], preferred_element_type=jnp.float32)
         + bias_ref[...] + x)
    g = 0.5 * h * (1.0 + lax.erf(h * _SQRT1_2))
    mean = jnp.mean(g, axis=-1, keepdims=True)
    cent = g - mean
    var = jnp.mean(cent * cent, axis=-1, keepdims=True)
    out_ref[...] = cent * lax.rsqrt(var + 1e-5) * g_ref[...] + b_ref[...]


def _finalize(partials, node_feats, loop_weight, bias, ln_gamma, ln_beta):
    nblk = N // XB
    return pl.pallas_call(
        _final_body,
        grid=(nblk,),
        in_specs=[
            pl.BlockSpec((NC, XB, F), lambda j: (0, j, 0)),
            pl.BlockSpec((XB, F), lambda j: (j, 0)),
            pl.BlockSpec((F, F), lambda j: (0, 0)),
            pl.BlockSpec((1, F), lambda j: (0, 0)),
            pl.BlockSpec((1, F), lambda j: (0, 0)),
            pl.BlockSpec((1, F), lambda j: (0, 0)),
        ],
        out_specs=pl.BlockSpec((XB, F), lambda j: (j, 0)),
        out_shape=jax.ShapeDtypeStruct((N, F), jnp.float32),
    )(partials, node_feats, loop_weight, bias.reshape(1, F),
      ln_gamma.reshape(1, F), ln_beta.reshape(1, F))


def kernel(node_feats, edge_index, edge_types, basis, w_comp, loop_weight,
           bias, ln_gamma, ln_beta):
    src = edge_index[0]
    dst = edge_index[1]
    pad = EP - E
    pad_iota = jnp.arange(pad, dtype=jnp.int32)
    src_p = jnp.concatenate([src, pad_iota % N]).reshape(EPR, F)
    et_p = jnp.concatenate([edge_types, jnp.zeros((pad,), jnp.int32)]
                           ).reshape(EPR, F)
    dst_p = jnp.concatenate([dst, N + pad_iota % (NPAD - N)]).reshape(EPR, F)

    all_t, keys = _all_transform(node_feats, basis, w_comp, src_p, et_p)
    partials = _build_sc_agg()(all_t, keys, dst_p)
    return _finalize(partials, node_feats, loop_weight, bias, ln_gamma, ln_beta)
